# baseline (speedup 1.0000x reference)
import jax
import jax.numpy as jnp
from jax.experimental import pallas as pl

_TM = 1000
_TM0 = 400
_KS = 2560


def _elu(x):
    return jnp.where(x > 0.0, x, jnp.exp(x) - 1.0)


def _first_body(pos_ref, w_ref, ol_ref, or_ref):
    s = jnp.dot(pos_ref[...], w_ref[...], preferred_element_type=jnp.float32)
    sl = ol_ref.shape[1]
    ol_ref[...] = s[:, :sl].astype(jnp.bfloat16)
    or_ref[...] = s[:, sl:]


def _tail(agg, sright_ref, bl_ref, br_ref, wtop_ref, wbot_ref, ol_ref, or_ref):
    xl = _elu(agg + bl_ref[...])
    xr = _elu(sright_ref[...] + br_ref[...])
    s = (jnp.dot(xl, wtop_ref[...], preferred_element_type=jnp.float32)
         + jnp.dot(xr, wbot_ref[...], preferred_element_type=jnp.float32))
    sln = ol_ref.shape[1]
    ol_ref[...] = s[:, :sln].astype(jnp.bfloat16)
    or_ref[...] = s[:, sln:]


def _aggn(ab_refs, sleft_ref):
    agg = None
    o = 0
    for ar in ab_refs:
        w = ar.shape[1]
        d = jnp.dot(ar[...], sleft_ref[o:o + w, :],
                    preferred_element_type=jnp.float32)
        agg = d if agg is None else agg + d
        o += w
    return agg


def _l0_body(*refs):
    adj_ref, sleft_ref, sright_ref, bl_ref, br_ref, wtop_ref, wbot_ref = refs[:7]
    ab_refs = refs[7:-2]
    ol_ref, or_ref = refs[-2:]
    ab = adj_ref[...].astype(jnp.bfloat16)
    o = 0
    for ar in ab_refs:
        w = ar.shape[1]
        ar[...] = ab[:, o:o + w]
        o += w
    agg = jnp.dot(ab, sleft_ref[...], preferred_element_type=jnp.float32)
    _tail(agg, sright_ref, bl_ref, br_ref, wtop_ref, wbot_ref, ol_ref, or_ref)


def _mid_body(*refs):
    sleft_ref, sright_ref, bl_ref, br_ref, wtop_ref, wbot_ref = refs[-8:-2]
    ol_ref, or_ref = refs[-2:]
    agg = _aggn(refs[:-8], sleft_ref)
    _tail(agg, sright_ref, bl_ref, br_ref, wtop_ref, wbot_ref, ol_ref, or_ref)


def _last_body(*refs):
    sleft_ref, sright_ref, bl_ref, br_ref = refs[-6:-2]
    ml_ref, mr_ref = refs[-2:]
    i = pl.program_id(0)
    agg = _aggn(refs[:-6], sleft_ref)
    xl = _elu(agg + bl_ref[...])
    xr = _elu(sright_ref[...] + br_ref[...])
    pml = jnp.max(xl, axis=0, keepdims=True)
    pmr = jnp.max(xr, axis=0, keepdims=True)

    @pl.when(i == 0)
    def _():
        ml_ref[...] = pml
        mr_ref[...] = pmr

    @pl.when(i > 0)
    def _():
        ml_ref[...] = jnp.maximum(ml_ref[...], pml)
        mr_ref[...] = jnp.maximum(mr_ref[...], pmr)


def kernel(positions, adj, Ws, bs):
    n, fin0 = positions.shape
    fouts = [W.shape[1] for W in Ws]
    sls = [max(f // 3, 2) for f in fouts]
    nl = len(Ws)
    f0, s0 = fouts[0], sls[0]
    widths = []
    rem = n
    while rem > 0:
        w = min(_KS, rem)
        widths.append(w)
        rem -= w

    sleft, sright = pl.pallas_call(
        _first_body,
        grid=(1,),
        in_specs=[
            pl.BlockSpec((n, fin0), lambda i: (0, 0)),
            pl.BlockSpec((fin0, f0), lambda i: (0, 0)),
        ],
        out_specs=[
            pl.BlockSpec((n, s0), lambda i: (0, 0)),
            pl.BlockSpec((n, f0 - s0), lambda i: (0, 0)),
        ],
        out_shape=[
            jax.ShapeDtypeStruct((n, s0), jnp.bfloat16),
            jax.ShapeDtypeStruct((n, f0 - s0), jnp.float32),
        ],
    )(positions, Ws[0])

    abs_ = None
    for L in range(nl - 1):
        sl, fout = sls[L], fouts[L]
        wr = fout - sl
        sln, fn = sls[L + 1], fouts[L + 1]
        b = bs[L].reshape(1, fout)
        bl, br = b[:, :sl], b[:, sl:]
        wtop, wbot = Ws[L + 1][:sl], Ws[L + 1][sl:]
        tm = _TM0 if L == 0 else _TM
        common_specs = [
            pl.BlockSpec((n, sl), lambda i: (0, 0)),
            pl.BlockSpec((tm, wr), lambda i: (i, 0)),
            pl.BlockSpec((1, sl), lambda i: (0, 0)),
            pl.BlockSpec((1, wr), lambda i: (0, 0)),
            pl.BlockSpec((sl, fn), lambda i: (0, 0)),
            pl.BlockSpec((wr, fn), lambda i: (0, 0)),
        ]
        out_specs = [
            pl.BlockSpec((tm, sln), lambda i: (i, 0)),
            pl.BlockSpec((tm, fn - sln), lambda i: (i, 0)),
        ]
        out_shape = [
            jax.ShapeDtypeStruct((n, sln), jnp.bfloat16),
            jax.ShapeDtypeStruct((n, fn - sln), jnp.float32),
        ]
        if L == 0:
            res = pl.pallas_call(
                _l0_body,
                grid=(n // _TM0,),
                in_specs=[pl.BlockSpec((tm, n), lambda i: (i, 0))] + common_specs,
                out_specs=[pl.BlockSpec((_TM0, w), lambda i: (i, 0))
                           for w in widths] + out_specs,
                out_shape=[jax.ShapeDtypeStruct((n, w), jnp.bfloat16)
                           for w in widths] + out_shape,
            )(adj, sleft, sright, bl, br, wtop, wbot)
            abs_, (sleft, sright) = res[:-2], res[-2:]
        else:
            sleft, sright = pl.pallas_call(
                _mid_body,
                grid=(n // _TM,),
                in_specs=[pl.BlockSpec((tm, w), lambda i: (i, 0))
                          for w in widths] + common_specs,
                out_specs=out_specs,
                out_shape=out_shape,
            )(*abs_, sleft, sright, bl, br, wtop, wbot)

    sl, fout = sls[-1], fouts[-1]
    wr = fout - sl
    b = bs[-1].reshape(1, fout)
    bl, br = b[:, :sl], b[:, sl:]
    ml, mr = pl.pallas_call(
        _last_body,
        grid=(n // _TM,),
        in_specs=[pl.BlockSpec((_TM, w), lambda i: (i, 0)) for w in widths] + [
            pl.BlockSpec((n, sl), lambda i: (0, 0)),
            pl.BlockSpec((_TM, wr), lambda i: (i, 0)),
            pl.BlockSpec((1, sl), lambda i: (0, 0)),
            pl.BlockSpec((1, wr), lambda i: (0, 0)),
        ],
        out_specs=[
            pl.BlockSpec((1, sl), lambda i: (0, 0)),
            pl.BlockSpec((1, wr), lambda i: (0, 0)),
        ],
        out_shape=[
            jax.ShapeDtypeStruct((1, sl), jnp.float32),
            jax.ShapeDtypeStruct((1, wr), jnp.float32),
        ],
    )(*abs_, sleft, sright, bl, br)

    return jnp.concatenate([ml[0], mr[0]], axis=0)


# speedup vs baseline: 1.0079x; 1.0079x over previous
import jax
import jax.numpy as jnp
from jax.experimental import pallas as pl

_TM = 1000
_TM0 = 400
_KS = 5120


def _elu(x):
    return jnp.where(x > 0.0, x, jnp.exp(x) - 1.0)


def _first_body(pos_ref, w_ref, ol_ref, or_ref):
    s = jnp.dot(pos_ref[...], w_ref[...], preferred_element_type=jnp.float32)
    sl = ol_ref.shape[1]
    ol_ref[...] = s[:, :sl].astype(jnp.bfloat16)
    or_ref[...] = s[:, sl:]


def _tail(agg, sright_ref, bl_ref, br_ref, wtop_ref, wbot_ref, ol_ref, or_ref):
    xl = _elu(agg + bl_ref[...])
    xr = _elu(sright_ref[...] + br_ref[...])
    s = (jnp.dot(xl, wtop_ref[...], preferred_element_type=jnp.float32)
         + jnp.dot(xr, wbot_ref[...], preferred_element_type=jnp.float32))
    sln = ol_ref.shape[1]
    ol_ref[...] = s[:, :sln].astype(jnp.bfloat16)
    or_ref[...] = s[:, sln:]


def _aggn(ab_refs, sleft_ref):
    agg = None
    o = 0
    for ar in ab_refs:
        w = ar.shape[1]
        d = jnp.dot(ar[...], sleft_ref[o:o + w, :],
                    preferred_element_type=jnp.float32)
        agg = d if agg is None else agg + d
        o += w
    return agg


def _l0_body(*refs):
    adj_ref, sleft_ref, sright_ref, bl_ref, br_ref, wtop_ref, wbot_ref = refs[:7]
    ab_refs = refs[7:-2]
    ol_ref, or_ref = refs[-2:]
    ab = adj_ref[...].astype(jnp.bfloat16)
    o = 0
    for ar in ab_refs:
        w = ar.shape[1]
        ar[...] = ab[:, o:o + w]
        o += w
    agg = jnp.dot(ab, sleft_ref[...], preferred_element_type=jnp.float32)
    _tail(agg, sright_ref, bl_ref, br_ref, wtop_ref, wbot_ref, ol_ref, or_ref)


def _mid_body(*refs):
    sleft_ref, sright_ref, bl_ref, br_ref, wtop_ref, wbot_ref = refs[-8:-2]
    ol_ref, or_ref = refs[-2:]
    agg = _aggn(refs[:-8], sleft_ref)
    _tail(agg, sright_ref, bl_ref, br_ref, wtop_ref, wbot_ref, ol_ref, or_ref)


def _last_body(*refs):
    sleft_ref, sright_ref, bl_ref, br_ref = refs[-6:-2]
    ml_ref, mr_ref = refs[-2:]
    i = pl.program_id(0)
    agg = _aggn(refs[:-6], sleft_ref)
    xl = _elu(agg + bl_ref[...])
    xr = _elu(sright_ref[...] + br_ref[...])
    pml = jnp.max(xl, axis=0, keepdims=True)
    pmr = jnp.max(xr, axis=0, keepdims=True)

    @pl.when(i == 0)
    def _():
        ml_ref[...] = pml
        mr_ref[...] = pmr

    @pl.when(i > 0)
    def _():
        ml_ref[...] = jnp.maximum(ml_ref[...], pml)
        mr_ref[...] = jnp.maximum(mr_ref[...], pmr)


def kernel(positions, adj, Ws, bs):
    n, fin0 = positions.shape
    fouts = [W.shape[1] for W in Ws]
    sls = [max(f // 3, 2) for f in fouts]
    nl = len(Ws)
    f0, s0 = fouts[0], sls[0]
    widths = []
    rem = n
    while rem > 0:
        w = min(_KS, rem)
        widths.append(w)
        rem -= w

    sleft, sright = pl.pallas_call(
        _first_body,
        grid=(1,),
        in_specs=[
            pl.BlockSpec((n, fin0), lambda i: (0, 0)),
            pl.BlockSpec((fin0, f0), lambda i: (0, 0)),
        ],
        out_specs=[
            pl.BlockSpec((n, s0), lambda i: (0, 0)),
            pl.BlockSpec((n, f0 - s0), lambda i: (0, 0)),
        ],
        out_shape=[
            jax.ShapeDtypeStruct((n, s0), jnp.bfloat16),
            jax.ShapeDtypeStruct((n, f0 - s0), jnp.float32),
        ],
    )(positions, Ws[0])

    abs_ = None
    for L in range(nl - 1):
        sl, fout = sls[L], fouts[L]
        wr = fout - sl
        sln, fn = sls[L + 1], fouts[L + 1]
        b = bs[L].reshape(1, fout)
        bl, br = b[:, :sl], b[:, sl:]
        wtop, wbot = Ws[L + 1][:sl], Ws[L + 1][sl:]
        tm = _TM0 if L == 0 else _TM
        common_specs = [
            pl.BlockSpec((n, sl), lambda i: (0, 0)),
            pl.BlockSpec((tm, wr), lambda i: (i, 0)),
            pl.BlockSpec((1, sl), lambda i: (0, 0)),
            pl.BlockSpec((1, wr), lambda i: (0, 0)),
            pl.BlockSpec((sl, fn), lambda i: (0, 0)),
            pl.BlockSpec((wr, fn), lambda i: (0, 0)),
        ]
        out_specs = [
            pl.BlockSpec((tm, sln), lambda i: (i, 0)),
            pl.BlockSpec((tm, fn - sln), lambda i: (i, 0)),
        ]
        out_shape = [
            jax.ShapeDtypeStruct((n, sln), jnp.bfloat16),
            jax.ShapeDtypeStruct((n, fn - sln), jnp.float32),
        ]
        if L == 0:
            res = pl.pallas_call(
                _l0_body,
                grid=(n // _TM0,),
                in_specs=[pl.BlockSpec((tm, n), lambda i: (i, 0))] + common_specs,
                out_specs=[pl.BlockSpec((_TM0, w), lambda i: (i, 0))
                           for w in widths] + out_specs,
                out_shape=[jax.ShapeDtypeStruct((n, w), jnp.bfloat16)
                           for w in widths] + out_shape,
            )(adj, sleft, sright, bl, br, wtop, wbot)
            abs_, (sleft, sright) = res[:-2], res[-2:]
        else:
            sleft, sright = pl.pallas_call(
                _mid_body,
                grid=(n // _TM,),
                in_specs=[pl.BlockSpec((tm, w), lambda i: (i, 0))
                          for w in widths] + common_specs,
                out_specs=out_specs,
                out_shape=out_shape,
            )(*abs_, sleft, sright, bl, br, wtop, wbot)

    sl, fout = sls[-1], fouts[-1]
    wr = fout - sl
    b = bs[-1].reshape(1, fout)
    bl, br = b[:, :sl], b[:, sl:]
    ml, mr = pl.pallas_call(
        _last_body,
        grid=(n // _TM,),
        in_specs=[pl.BlockSpec((_TM, w), lambda i: (i, 0)) for w in widths] + [
            pl.BlockSpec((n, sl), lambda i: (0, 0)),
            pl.BlockSpec((_TM, wr), lambda i: (i, 0)),
            pl.BlockSpec((1, sl), lambda i: (0, 0)),
            pl.BlockSpec((1, wr), lambda i: (0, 0)),
        ],
        out_specs=[
            pl.BlockSpec((1, sl), lambda i: (0, 0)),
            pl.BlockSpec((1, wr), lambda i: (0, 0)),
        ],
        out_shape=[
            jax.ShapeDtypeStruct((1, sl), jnp.float32),
            jax.ShapeDtypeStruct((1, wr), jnp.float32),
        ],
    )(*abs_, sleft, sright, bl, br)

    return jnp.concatenate([ml[0], mr[0]], axis=0)


# speedup vs baseline: 1.0275x; 1.0195x over previous
import jax
import jax.numpy as jnp
from jax.experimental import pallas as pl

_TM = 1000
_TM0 = 400
_KS = 5120


def _elu(x):
    return jnp.where(x > 0.0, x, jnp.exp(x) - 1.0)


def _first_body(pos_ref, w_ref, ol_ref, or_ref):
    s = jnp.dot(pos_ref[...], w_ref[...], preferred_element_type=jnp.float32)
    sl = ol_ref.shape[1]
    ol_ref[...] = s[:, :sl].astype(jnp.bfloat16)
    or_ref[...] = s[:, sl:].astype(jnp.bfloat16)


def _tail(agg, sright_ref, bl_ref, br_ref, wtop_ref, wbot_ref, ol_ref, or_ref):
    xl = _elu(agg + bl_ref[...])
    xr = _elu(sright_ref[...] + br_ref[...])
    s = (jnp.dot(xl, wtop_ref[...], preferred_element_type=jnp.float32)
         + jnp.dot(xr, wbot_ref[...], preferred_element_type=jnp.float32))
    sln = ol_ref.shape[1]
    ol_ref[...] = s[:, :sln].astype(jnp.bfloat16)
    or_ref[...] = s[:, sln:].astype(jnp.bfloat16)


def _aggn(ab_refs, sleft_ref):
    agg = None
    o = 0
    for ar in ab_refs:
        w = ar.shape[1]
        d = jnp.dot(ar[...], sleft_ref[o:o + w, :],
                    preferred_element_type=jnp.float32)
        agg = d if agg is None else agg + d
        o += w
    return agg


def _l0_body(*refs):
    adj_ref, sleft_ref, sright_ref, bl_ref, br_ref, wtop_ref, wbot_ref = refs[:7]
    ab_refs = refs[7:-2]
    ol_ref, or_ref = refs[-2:]
    ab = adj_ref[...].astype(jnp.bfloat16)
    o = 0
    for ar in ab_refs:
        w = ar.shape[1]
        ar[...] = ab[:, o:o + w]
        o += w
    agg = jnp.dot(ab, sleft_ref[...], preferred_element_type=jnp.float32)
    _tail(agg, sright_ref, bl_ref, br_ref, wtop_ref, wbot_ref, ol_ref, or_ref)


def _mid_body(*refs):
    sleft_ref, sright_ref, bl_ref, br_ref, wtop_ref, wbot_ref = refs[-8:-2]
    ol_ref, or_ref = refs[-2:]
    agg = _aggn(refs[:-8], sleft_ref)
    _tail(agg, sright_ref, bl_ref, br_ref, wtop_ref, wbot_ref, ol_ref, or_ref)


def _last_body(*refs):
    sleft_ref, sright_ref, bl_ref, br_ref = refs[-6:-2]
    ml_ref, mr_ref = refs[-2:]
    i = pl.program_id(0)
    agg = _aggn(refs[:-6], sleft_ref)
    xl = _elu(agg + bl_ref[...])
    xr = _elu(sright_ref[...] + br_ref[...])
    pml = jnp.max(xl, axis=0, keepdims=True)
    pmr = jnp.max(xr, axis=0, keepdims=True)

    @pl.when(i == 0)
    def _():
        ml_ref[...] = pml
        mr_ref[...] = pmr

    @pl.when(i > 0)
    def _():
        ml_ref[...] = jnp.maximum(ml_ref[...], pml)
        mr_ref[...] = jnp.maximum(mr_ref[...], pmr)


def kernel(positions, adj, Ws, bs):
    n, fin0 = positions.shape
    fouts = [W.shape[1] for W in Ws]
    sls = [max(f // 3, 2) for f in fouts]
    nl = len(Ws)
    f0, s0 = fouts[0], sls[0]
    widths = []
    rem = n
    while rem > 0:
        w = min(_KS, rem)
        widths.append(w)
        rem -= w

    sleft, sright = pl.pallas_call(
        _first_body,
        grid=(1,),
        in_specs=[
            pl.BlockSpec((n, fin0), lambda i: (0, 0)),
            pl.BlockSpec((fin0, f0), lambda i: (0, 0)),
        ],
        out_specs=[
            pl.BlockSpec((n, s0), lambda i: (0, 0)),
            pl.BlockSpec((n, f0 - s0), lambda i: (0, 0)),
        ],
        out_shape=[
            jax.ShapeDtypeStruct((n, s0), jnp.bfloat16),
            jax.ShapeDtypeStruct((n, f0 - s0), jnp.bfloat16),
        ],
    )(positions, Ws[0])

    abs_ = None
    for L in range(nl - 1):
        sl, fout = sls[L], fouts[L]
        wr = fout - sl
        sln, fn = sls[L + 1], fouts[L + 1]
        b = bs[L].reshape(1, fout)
        bl, br = b[:, :sl], b[:, sl:]
        wtop, wbot = Ws[L + 1][:sl], Ws[L + 1][sl:]
        tm = _TM0 if L == 0 else _TM
        common_specs = [
            pl.BlockSpec((n, sl), lambda i: (0, 0)),
            pl.BlockSpec((tm, wr), lambda i: (i, 0)),
            pl.BlockSpec((1, sl), lambda i: (0, 0)),
            pl.BlockSpec((1, wr), lambda i: (0, 0)),
            pl.BlockSpec((sl, fn), lambda i: (0, 0)),
            pl.BlockSpec((wr, fn), lambda i: (0, 0)),
        ]
        out_specs = [
            pl.BlockSpec((tm, sln), lambda i: (i, 0)),
            pl.BlockSpec((tm, fn - sln), lambda i: (i, 0)),
        ]
        out_shape = [
            jax.ShapeDtypeStruct((n, sln), jnp.bfloat16),
            jax.ShapeDtypeStruct((n, fn - sln), jnp.bfloat16),
        ]
        if L == 0:
            res = pl.pallas_call(
                _l0_body,
                grid=(n // _TM0,),
                in_specs=[pl.BlockSpec((tm, n), lambda i: (i, 0))] + common_specs,
                out_specs=[pl.BlockSpec((_TM0, w), lambda i: (i, 0))
                           for w in widths] + out_specs,
                out_shape=[jax.ShapeDtypeStruct((n, w), jnp.bfloat16)
                           for w in widths] + out_shape,
            )(adj, sleft, sright, bl, br, wtop, wbot)
            abs_, (sleft, sright) = res[:-2], res[-2:]
        else:
            sleft, sright = pl.pallas_call(
                _mid_body,
                grid=(n // _TM,),
                in_specs=[pl.BlockSpec((tm, w), lambda i: (i, 0))
                          for w in widths] + common_specs,
                out_specs=out_specs,
                out_shape=out_shape,
            )(*abs_, sleft, sright, bl, br, wtop, wbot)

    sl, fout = sls[-1], fouts[-1]
    wr = fout - sl
    b = bs[-1].reshape(1, fout)
    bl, br = b[:, :sl], b[:, sl:]
    ml, mr = pl.pallas_call(
        _last_body,
        grid=(n // _TM,),
        in_specs=[pl.BlockSpec((_TM, w), lambda i: (i, 0)) for w in widths] + [
            pl.BlockSpec((n, sl), lambda i: (0, 0)),
            pl.BlockSpec((_TM, wr), lambda i: (i, 0)),
            pl.BlockSpec((1, sl), lambda i: (0, 0)),
            pl.BlockSpec((1, wr), lambda i: (0, 0)),
        ],
        out_specs=[
            pl.BlockSpec((1, sl), lambda i: (0, 0)),
            pl.BlockSpec((1, wr), lambda i: (0, 0)),
        ],
        out_shape=[
            jax.ShapeDtypeStruct((1, sl), jnp.float32),
            jax.ShapeDtypeStruct((1, wr), jnp.float32),
        ],
    )(*abs_, sleft, sright, bl, br)

    return jnp.concatenate([ml[0], mr[0]], axis=0)


# speedup vs baseline: 1.0308x; 1.0032x over previous
import jax
import jax.numpy as jnp
from jax.experimental import pallas as pl

_TM = 1000
_TM0 = 400
_KS = 5120


def _elu(x):
    return jnp.where(x > 0.0, x, jnp.exp(x) - 1.0)


def _first_body(pos_ref, w_ref, ol_ref, or_ref):
    s = jnp.dot(pos_ref[...], w_ref[...], preferred_element_type=jnp.float32)
    sl = ol_ref.shape[1]
    ol_ref[...] = s[:, :sl].astype(jnp.bfloat16)
    or_ref[...] = s[:, sl:].astype(jnp.bfloat16)


def _tail(agg, sright_ref, bl_ref, br_ref, wtop_ref, wbot_ref, ol_ref, or_ref):
    xl = _elu(agg + bl_ref[...])
    xr = _elu(sright_ref[...] + br_ref[...])
    s = (jnp.dot(xl, wtop_ref[...], preferred_element_type=jnp.float32)
         + jnp.dot(xr, wbot_ref[...], preferred_element_type=jnp.float32))
    sln = ol_ref.shape[1]
    ol_ref[...] = s[:, :sln].astype(jnp.bfloat16)
    or_ref[...] = s[:, sln:].astype(jnp.bfloat16)


def _aggn(ab_refs, sleft_ref):
    agg = None
    o = 0
    for ar in ab_refs:
        w = ar.shape[1]
        d = jnp.dot(ar[...], sleft_ref[o:o + w, :],
                    preferred_element_type=jnp.float32)
        agg = d if agg is None else agg + d
        o += w
    return agg


def _l0_body(*refs):
    adj1_ref, adj2_ref, sleft_ref, sright_ref, bl_ref, br_ref, wtop_ref, \
        wbot_ref = refs[:8]
    ab1_ref, ab2_ref, ol_ref, or_ref = refs[8:]
    w1 = ab1_ref.shape[1]
    w2 = ab2_ref.shape[1]
    ab1 = adj1_ref[...].astype(jnp.bfloat16)
    ab2 = adj2_ref[:, :w2].astype(jnp.bfloat16)
    ab1_ref[...] = ab1
    ab2_ref[...] = ab2
    agg = (jnp.dot(ab1, sleft_ref[:w1, :], preferred_element_type=jnp.float32)
           + jnp.dot(ab2, sleft_ref[w1:, :], preferred_element_type=jnp.float32))
    _tail(agg, sright_ref, bl_ref, br_ref, wtop_ref, wbot_ref, ol_ref, or_ref)


def _mid_body(*refs):
    sleft_ref, sright_ref, bl_ref, br_ref, wtop_ref, wbot_ref = refs[-8:-2]
    ol_ref, or_ref = refs[-2:]
    agg = _aggn(refs[:-8], sleft_ref)
    _tail(agg, sright_ref, bl_ref, br_ref, wtop_ref, wbot_ref, ol_ref, or_ref)


def _last_body(*refs):
    sleft_ref, sright_ref, bl_ref, br_ref = refs[-6:-2]
    ml_ref, mr_ref = refs[-2:]
    i = pl.program_id(0)
    agg = _aggn(refs[:-6], sleft_ref)
    xl = _elu(agg + bl_ref[...])
    xr = _elu(sright_ref[...] + br_ref[...])
    pml = jnp.max(xl, axis=0, keepdims=True)
    pmr = jnp.max(xr, axis=0, keepdims=True)

    @pl.when(i == 0)
    def _():
        ml_ref[...] = pml
        mr_ref[...] = pmr

    @pl.when(i > 0)
    def _():
        ml_ref[...] = jnp.maximum(ml_ref[...], pml)
        mr_ref[...] = jnp.maximum(mr_ref[...], pmr)


def kernel(positions, adj, Ws, bs):
    n, fin0 = positions.shape
    fouts = [W.shape[1] for W in Ws]
    sls = [max(f // 3, 2) for f in fouts]
    nl = len(Ws)
    f0, s0 = fouts[0], sls[0]
    widths = []
    rem = n
    while rem > 0:
        w = min(_KS, rem)
        widths.append(w)
        rem -= w

    sleft, sright = pl.pallas_call(
        _first_body,
        grid=(1,),
        in_specs=[
            pl.BlockSpec((n, fin0), lambda i: (0, 0)),
            pl.BlockSpec((fin0, f0), lambda i: (0, 0)),
        ],
        out_specs=[
            pl.BlockSpec((n, s0), lambda i: (0, 0)),
            pl.BlockSpec((n, f0 - s0), lambda i: (0, 0)),
        ],
        out_shape=[
            jax.ShapeDtypeStruct((n, s0), jnp.bfloat16),
            jax.ShapeDtypeStruct((n, f0 - s0), jnp.bfloat16),
        ],
    )(positions, Ws[0])

    abs_ = None
    for L in range(nl - 1):
        sl, fout = sls[L], fouts[L]
        wr = fout - sl
        sln, fn = sls[L + 1], fouts[L + 1]
        b = bs[L].reshape(1, fout)
        bl, br = b[:, :sl], b[:, sl:]
        wtop, wbot = Ws[L + 1][:sl], Ws[L + 1][sl:]
        tm = _TM0 if L == 0 else _TM
        common_specs = [
            pl.BlockSpec((n, sl), lambda i: (0, 0)),
            pl.BlockSpec((tm, wr), lambda i: (i, 0)),
            pl.BlockSpec((1, sl), lambda i: (0, 0)),
            pl.BlockSpec((1, wr), lambda i: (0, 0)),
            pl.BlockSpec((sl, fn), lambda i: (0, 0)),
            pl.BlockSpec((wr, fn), lambda i: (0, 0)),
        ]
        out_specs = [
            pl.BlockSpec((tm, sln), lambda i: (i, 0)),
            pl.BlockSpec((tm, fn - sln), lambda i: (i, 0)),
        ]
        out_shape = [
            jax.ShapeDtypeStruct((n, sln), jnp.bfloat16),
            jax.ShapeDtypeStruct((n, fn - sln), jnp.bfloat16),
        ]
        if L == 0:
            res = pl.pallas_call(
                _l0_body,
                grid=(n // _TM0,),
                in_specs=[pl.BlockSpec((tm, widths[0]), lambda i: (i, 0)),
                          pl.BlockSpec((tm, widths[0]), lambda i: (i, 1))]
                         + common_specs,
                out_specs=[pl.BlockSpec((_TM0, w), lambda i: (i, 0))
                           for w in widths] + out_specs,
                out_shape=[jax.ShapeDtypeStruct((n, w), jnp.bfloat16)
                           for w in widths] + out_shape,
            )(adj, adj, sleft, sright, bl, br, wtop, wbot)
            abs_, (sleft, sright) = res[:-2], res[-2:]
        else:
            sleft, sright = pl.pallas_call(
                _mid_body,
                grid=(n // _TM,),
                in_specs=[pl.BlockSpec((tm, w), lambda i: (i, 0))
                          for w in widths] + common_specs,
                out_specs=out_specs,
                out_shape=out_shape,
            )(*abs_, sleft, sright, bl, br, wtop, wbot)

    sl, fout = sls[-1], fouts[-1]
    wr = fout - sl
    b = bs[-1].reshape(1, fout)
    bl, br = b[:, :sl], b[:, sl:]
    ml, mr = pl.pallas_call(
        _last_body,
        grid=(n // _TM,),
        in_specs=[pl.BlockSpec((_TM, w), lambda i: (i, 0)) for w in widths] + [
            pl.BlockSpec((n, sl), lambda i: (0, 0)),
            pl.BlockSpec((_TM, wr), lambda i: (i, 0)),
            pl.BlockSpec((1, sl), lambda i: (0, 0)),
            pl.BlockSpec((1, wr), lambda i: (0, 0)),
        ],
        out_specs=[
            pl.BlockSpec((1, sl), lambda i: (0, 0)),
            pl.BlockSpec((1, wr), lambda i: (0, 0)),
        ],
        out_shape=[
            jax.ShapeDtypeStruct((1, sl), jnp.float32),
            jax.ShapeDtypeStruct((1, wr), jnp.float32),
        ],
    )(*abs_, sleft, sright, bl, br)

    return jnp.concatenate([ml[0], mr[0]], axis=0)
